# baseline (speedup 1.0000x reference)
import functools

import jax
import jax.numpy as jnp
from jax import lax
from jax.experimental import pallas as pl
from jax.experimental.pallas import tpu as pltpu
from jax.experimental.pallas import tpu_sc as plsc

BATCH = 16384
DIM = 16
LANES = 16
ROWS_PER_SUPER = 128 // DIM
SUPER = 125000
NUM_CORES = 2
NUM_SUBCORES = 16
NUM_WORKERS = NUM_CORES * NUM_SUBCORES
BPW = BATCH // NUM_WORKERS
GC = 128
NCHUNK = BPW // GC
VPC = GC // LANES


def _neumf_body(uidx_hbm, iidx_hbm, umf_hbm, imf_hbm, umlp_hbm, imlp_hbm,
                params_hbm, out_hbm,
                uidx_v, iidx_v, usr_v, isr_v, umf_v, imf_v, umlp_v, imlp_v,
                params_v, out_v, sem):
    wid = lax.axis_index("s") * NUM_CORES + lax.axis_index("c")
    base = wid * BPW
    pltpu.sync_copy(params_hbm, params_v)
    pltpu.sync_copy(uidx_hbm.at[pl.ds(base, BPW)], uidx_v)
    pltpu.sync_copy(iidx_hbm.at[pl.ds(base, BPW)], iidx_v)

    def sr(i, carry):
        b0 = i * LANES
        uv = uidx_v[pl.ds(b0, LANES)]
        iv = iidx_v[pl.ds(b0, LANES)]
        usr_v[pl.ds(b0, LANES)] = lax.shift_right_logical(uv, 3)
        isr_v[pl.ds(b0, LANES)] = lax.shift_right_logical(iv, 3)
        return carry

    lax.fori_loop(0, BPW // LANES, sr, 0)

    w_mlp = params_v[pl.ds(0, LANES)]
    w_mf = params_v[pl.ds(DIM, LANES)]
    bias = params_v[pl.ds(2 * DIM, LANES)][0]

    def chunk(c, carry):
        g0 = c * GC
        c1 = pltpu.async_copy(umf_hbm.at[usr_v.at[pl.ds(g0, GC)]], umf_v, sem)
        c2 = pltpu.async_copy(imf_hbm.at[isr_v.at[pl.ds(g0, GC)]], imf_v, sem)
        c3 = pltpu.async_copy(umlp_hbm.at[usr_v.at[pl.ds(g0, GC)]], umlp_v, sem)
        c4 = pltpu.async_copy(imlp_hbm.at[isr_v.at[pl.ds(g0, GC)]], imlp_v, sem)
        c1.wait()
        c2.wait()
        c3.wait()
        c4.wait()

        def vec(v, carry2):
            b0 = v * LANES
            rows = b0 + lax.iota(jnp.int32, LANES)
            uoff = (uidx_v[pl.ds(g0 + b0, LANES)] & (ROWS_PER_SUPER - 1)) * DIM
            ioff = (iidx_v[pl.ds(g0 + b0, LANES)] & (ROWS_PER_SUPER - 1)) * DIM
            acc = jnp.full((LANES,), bias, jnp.float32)
            for d in range(DIM):
                gu = plsc.load_gather(umlp_v, [rows, uoff + d])
                gi = plsc.load_gather(imlp_v, [rows, ioff + d])
                acc = acc + (gu * gi) * w_mlp[d]
                gu2 = plsc.load_gather(umf_v, [rows, uoff + d])
                gi2 = plsc.load_gather(imf_v, [rows, ioff + d])
                acc = acc + (gu2 * gi2) * w_mf[d]
            out_v[pl.ds(g0 + b0, LANES)] = 1.0 / (1.0 + jnp.exp(-acc))
            return carry2

        lax.fori_loop(0, VPC, vec, 0)
        return carry

    lax.fori_loop(0, NCHUNK, chunk, 0)
    pltpu.sync_copy(out_v, out_hbm.at[pl.ds(base, BPW)])


@jax.jit
def kernel(user_indices, item_indices, emb_user_mf, emb_item_mf,
           emb_user_mlp, emb_item_mlp, affine_w, affine_b):
    params = jnp.concatenate(
        [affine_w[0], affine_b, jnp.zeros((15,), jnp.float32)])
    mesh = plsc.VectorSubcoreMesh(core_axis_name="c", subcore_axis_name="s")
    run = functools.partial(
        pl.kernel,
        mesh=mesh,
        compiler_params=pltpu.CompilerParams(
            needs_layout_passes=False, use_tc_tiling_on_sc=False),
        out_type=jax.ShapeDtypeStruct((BATCH,), jnp.float32),
        scratch_types=[
            pltpu.VMEM((BPW,), jnp.int32),
            pltpu.VMEM((BPW,), jnp.int32),
            pltpu.VMEM((BPW,), jnp.int32),
            pltpu.VMEM((BPW,), jnp.int32),
            pltpu.VMEM((GC, 128), jnp.float32),
            pltpu.VMEM((GC, 128), jnp.float32),
            pltpu.VMEM((GC, 128), jnp.float32),
            pltpu.VMEM((GC, 128), jnp.float32),
            pltpu.VMEM((2 * DIM + 16,), jnp.float32),
            pltpu.VMEM((BPW,), jnp.float32),
            pltpu.SemaphoreType.DMA,
        ],
    )(_neumf_body)
    out = run(user_indices.astype(jnp.int32), item_indices.astype(jnp.int32),
              emb_user_mf.reshape(SUPER, 128), emb_item_mf.reshape(SUPER, 128),
              emb_user_mlp.reshape(SUPER, 128),
              emb_item_mlp.reshape(SUPER, 128), params)
    return out.reshape(BATCH, 1)


# speedup vs baseline: 6.2733x; 6.2733x over previous
import functools

import jax
import jax.numpy as jnp
from jax import lax
from jax.experimental import pallas as pl
from jax.experimental.pallas import tpu as pltpu
from jax.experimental.pallas import tpu_sc as plsc

BATCH = 16384
DIM = 16
LANES = 16
NUM_ROWS = 1000000
NUM_CORES = 2
NUM_SUBCORES = 16
NUM_WORKERS = NUM_CORES * NUM_SUBCORES
BPW = BATCH // NUM_WORKERS
CHUNKS = BPW // LANES


def _neumf_body(uidx_hbm, iidx_hbm, umf_hbm, imf_hbm, umlp_hbm, imlp_hbm,
                params_hbm, out_hbm,
                uidx_v, iidx_v, ubuf_v, ibuf_v, params_v, out_v, sem):
    wid = lax.axis_index("s") * NUM_CORES + lax.axis_index("c")
    base = wid * BPW
    pltpu.sync_copy(params_hbm, params_v)
    pltpu.sync_copy(uidx_hbm.at[pl.ds(base, BPW)], uidx_v)
    pltpu.sync_copy(iidx_hbm.at[pl.ds(base, BPW)], iidx_v)

    w_mlp = params_v[pl.ds(0, LANES)]
    w_mf = params_v[pl.ds(DIM, LANES)]
    bias = params_v[pl.ds(2 * DIM, LANES)][0]

    lanes = lax.iota(jnp.int32, LANES)

    def half(utbl, itbl, w, uv, iv, acc):
        copies = []
        for j in range(LANES):
            uoff = pl.multiple_of(
                lax.shift_left(lax.shift_right_logical(uv[j], 7), 7), 128)
            ioff = pl.multiple_of(
                lax.shift_left(lax.shift_right_logical(iv[j], 7), 7), 128)
            copies.append(pltpu.async_copy(
                utbl.at[:, pl.ds(uoff, 128)], ubuf_v.at[j], sem))
            copies.append(pltpu.async_copy(
                itbl.at[:, pl.ds(ioff, 128)], ibuf_v.at[j], sem))
        for cp in copies:
            cp.wait()
        uln = uv & 127
        iln = iv & 127
        for d in range(DIM):
            drow = jnp.full((LANES,), d, jnp.int32)
            gu = plsc.load_gather(ubuf_v, [lanes, drow, uln])
            gi = plsc.load_gather(ibuf_v, [lanes, drow, iln])
            acc = acc + (gu * gi) * w[d]
        return acc

    def chunk(c, carry):
        b0 = c * LANES
        uv = uidx_v[pl.ds(b0, LANES)]
        iv = iidx_v[pl.ds(b0, LANES)]
        acc = jnp.full((LANES,), bias, jnp.float32)
        acc = half(umlp_hbm, imlp_hbm, w_mlp, uv, iv, acc)
        acc = half(umf_hbm, imf_hbm, w_mf, uv, iv, acc)
        out_v[pl.ds(b0, LANES)] = 1.0 / (1.0 + jnp.exp(-acc))
        return carry

    lax.fori_loop(0, CHUNKS, chunk, 0)
    pltpu.sync_copy(out_v, out_hbm.at[pl.ds(base, BPW)])


@jax.jit
def kernel(user_indices, item_indices, emb_user_mf, emb_item_mf,
           emb_user_mlp, emb_item_mlp, affine_w, affine_b):
    params = jnp.concatenate(
        [affine_w[0], affine_b, jnp.zeros((15,), jnp.float32)])
    mesh = plsc.VectorSubcoreMesh(core_axis_name="c", subcore_axis_name="s")
    run = functools.partial(
        pl.kernel,
        mesh=mesh,
        compiler_params=pltpu.CompilerParams(
            needs_layout_passes=False, use_tc_tiling_on_sc=True),
        out_type=jax.ShapeDtypeStruct((BATCH,), jnp.float32),
        scratch_types=[
            pltpu.VMEM((BPW,), jnp.int32),
            pltpu.VMEM((BPW,), jnp.int32),
            pltpu.VMEM((LANES, DIM, 128), jnp.float32),
            pltpu.VMEM((LANES, DIM, 128), jnp.float32),
            pltpu.VMEM((2 * DIM + 16,), jnp.float32),
            pltpu.VMEM((BPW,), jnp.float32),
            pltpu.SemaphoreType.DMA,
        ],
    )(_neumf_body)
    out = run(user_indices.astype(jnp.int32), item_indices.astype(jnp.int32),
              emb_user_mf.T, emb_item_mf.T, emb_user_mlp.T, emb_item_mlp.T,
              params)
    return out.reshape(BATCH, 1)
